# Initial kernel scaffold; baseline (speedup 1.0000x reference)
import jax
import jax.numpy as jnp
from jax.experimental import pallas as pl


def kernel(z_e, W_down, W_up, codebook):
    raise NotImplementedError("write your pallas kernel here")


# speedup vs baseline: 1.3244x; 1.3244x over previous
import functools

import jax
import jax.numpy as jnp
from jax import lax
from jax.experimental import pallas as pl
from jax.experimental.pallas import tpu as pltpu
from jax.experimental.pallas import tpu_sc as plsc


def _cup_body(cb_ref, wu_ref, out_ref):
    out_ref[...] = jnp.dot(cb_ref[...], wu_ref[...],
                           preferred_element_type=jnp.float32)


def _project_codebook(codebook, wu_t):
    K, _ = codebook.shape
    D = wu_t.shape[1]
    return pl.pallas_call(
        _cup_body,
        out_shape=jax.ShapeDtypeStruct((K, D), jnp.float32),
    )(codebook, wu_t)


def _vq_body(z_ref, wd_ref, ct_ref, zd_ref, code_ref, loss_ref):
    T, DC = zd_ref.shape[1], zd_ref.shape[2]
    z = z_ref[0]
    zd = jnp.dot(z, wd_ref[...], preferred_element_type=jnp.float32)
    zd_ref[0] = zd
    ct = ct_ref[...]
    scores = jnp.dot(zd, ct, preferred_element_type=jnp.float32)
    cnorm = jnp.sum(ct * ct, axis=0, keepdims=True)
    znorm = jnp.sum(zd * zd, axis=1, keepdims=True)
    dist = znorm - 2.0 * scores + cnorm
    code_ref[0, 0] = jnp.argmin(dist, axis=1).astype(jnp.int32)
    loss = jnp.sum(jnp.min(dist, axis=1)) * (1.0 / (T * DC))
    loss_ref[0, 0] = jnp.full((128,), loss, jnp.float32)


def _vq_quantize(z_e, wd_t, ct):
    B, T, DIN = z_e.shape
    DC, K = ct.shape
    return pl.pallas_call(
        _vq_body,
        grid=(B,),
        in_specs=[
            pl.BlockSpec((1, T, DIN), lambda b: (b, 0, 0)),
            pl.BlockSpec((DIN, DC), lambda b: (0, 0)),
            pl.BlockSpec((DC, K), lambda b: (0, 0)),
        ],
        out_specs=[
            pl.BlockSpec((1, T, DC), lambda b: (b, 0, 0)),
            pl.BlockSpec((1, 1, T), lambda b: (b, 0, 0)),
            pl.BlockSpec((1, 1, 128), lambda b: (b, 0, 0)),
        ],
        out_shape=[
            jax.ShapeDtypeStruct((B, T, DC), jnp.float32),
            jax.ShapeDtypeStruct((B, 1, T), jnp.int32),
            jax.ShapeDtypeStruct((B, 1, 128), jnp.float32),
        ],
    )(z_e, wd_t, ct)


def _sc_gather(cup, code_flat):
    info = plsc.get_sparse_core_info()
    NC, NS = info.num_cores, info.num_subcores
    NW = NC * NS
    n = code_flat.shape[0]
    D = cup.shape[1]
    bpw = n // NW
    CH = 96
    mesh = plsc.VectorSubcoreMesh(core_axis_name="c", subcore_axis_name="s")

    @functools.partial(
        pl.kernel,
        mesh=mesh,
        out_type=jax.ShapeDtypeStruct((n, D), jnp.float32),
        scratch_types=[
            pltpu.VMEM((bpw,), jnp.int32),
            pltpu.VMEM((CH, D), jnp.float32),
            pltpu.SemaphoreType.DMA,
        ],
    )
    def k(cup_hbm, idx_hbm, out_hbm, idx_v, rows_v, sem):
        wid = lax.axis_index("s") * NC + lax.axis_index("c")
        base = wid * bpw
        pltpu.sync_copy(idx_hbm.at[pl.ds(base, bpw)], idx_v)
        for j in range(bpw // CH):
            pltpu.async_copy(cup_hbm.at[idx_v.at[pl.ds(j * CH, CH)]],
                             rows_v, sem).wait()
            pltpu.sync_copy(rows_v, out_hbm.at[pl.ds(base + j * CH, CH)])

    return k(cup, code_flat)


def kernel(z_e, W_down, W_up, codebook):
    B, T, DIN = z_e.shape
    wd_t = W_down.T
    ct = codebook.T
    wu_t = W_up.T

    cup = _project_codebook(codebook, wu_t)
    zd, code3, loss3 = _vq_quantize(z_e, wd_t, ct)
    code = code3.reshape(B, T)
    loss = loss3[:, 0, 0]
    zq_flat = _sc_gather(cup, code3.reshape(B * T))
    z_q = zq_flat.reshape(B, T, DIN)
    return (z_q, zd, code, loss, loss)
